# Initial kernel scaffold; baseline (speedup 1.0000x reference)
import jax
import jax.numpy as jnp
from jax.experimental import pallas as pl


def kernel(imu_seq, W1, b1, W2, b2, gate):
    raise NotImplementedError("write your pallas kernel here")


# speedup vs baseline: 9.7367x; 9.7367x over previous
import functools

import jax
import jax.numpy as jnp
from jax.experimental import pallas as pl

B, T, DIN, DH, DM, K = 16, 4096, 32, 64, 128, 32
SEG = T // K


def _mlp_pool_kernel(x_ref, w1_ref, b1_ref, w2_ref, b2_ref, gate_ref, o_ref):
    x = x_ref[0]
    h = jnp.dot(x, w1_ref[...], preferred_element_type=jnp.float32) + b1_ref[...]
    h = 0.5 * h * (1.0 + jax.lax.erf(h * jnp.float32(0.7071067811865476)))
    row = jax.lax.broadcasted_iota(jnp.int32, (K, T), 0)
    col = jax.lax.broadcasted_iota(jnp.int32, (K, T), 1)
    p = jnp.where(col // SEG == row, 1.0 / SEG, 0.0).astype(jnp.float32)
    s = jnp.dot(p, h, preferred_element_type=jnp.float32)
    y = jnp.dot(s, w2_ref[...], preferred_element_type=jnp.float32) + b2_ref[...]
    o_ref[0] = y * jnp.tanh(gate_ref[0, 0])


def kernel(imu_seq, W1, b1, W2, b2, gate):
    b1r = b1.reshape(1, DH)
    b2r = b2.reshape(1, DM)
    gr = gate.reshape(1, 1)
    out = pl.pallas_call(
        _mlp_pool_kernel,
        grid=(B,),
        in_specs=[
            pl.BlockSpec((1, T, DIN), lambda b: (b, 0, 0)),
            pl.BlockSpec((DIN, DH), lambda b: (0, 0)),
            pl.BlockSpec((1, DH), lambda b: (0, 0)),
            pl.BlockSpec((DH, DM), lambda b: (0, 0)),
            pl.BlockSpec((1, DM), lambda b: (0, 0)),
            pl.BlockSpec((1, 1), lambda b: (0, 0)),
        ],
        out_specs=pl.BlockSpec((1, K, DM), lambda b: (b, 0, 0)),
        out_shape=jax.ShapeDtypeStruct((B, K, DM), jnp.float32),
    )(imu_seq, W1, b1r, W2, b2r, gr)
    return out
